# baseline (speedup 1.0000x reference)
import jax
import jax.numpy as jnp
from jax import lax
from jax.experimental import pallas as pl
from jax.experimental.pallas import tpu as pltpu
from jax.experimental.pallas import tpu_sc as plsc

N = 10000
E = 320000
D = 128
TEMP = 0.1
EPS = 1e-20

NP = 10240
SROWS = 79
EP = 32 * SROWS * 128
BROWS = 2 * SROWS
NPT = NP // 16
DH = D // 2
UROWS = EP // 128

_NEG = -1e30

_SC_PARAMS = pltpu.CompilerParams(needs_layout_passes=False,
                                  use_tc_tiling_on_sc=False)
_SC_MESH = plsc.VectorSubcoreMesh(core_axis_name="c", subcore_axis_name="s")


def _tc1_body(x_ref, w_ref, u_ref, s_ref, g_ref):
    s_ref[...] = jnp.dot(x_ref[...], w_ref[...],
                         preferred_element_type=jnp.float32)
    u = u_ref[...]
    g_ref[...] = -jnp.log(-jnp.log(u + EPS) + EPS)


_tc1_stage = pl.pallas_call(
    _tc1_body,
    out_shape=(
        jax.ShapeDtypeStruct((N, 8), jnp.float32),
        jax.ShapeDtypeStruct((UROWS, 128), jnp.float32),
    ),
)


def _tc2_body(pub_ref, w_ref):
    m_w = pub_ref[:, 0, :]
    s_w = pub_ref[:, 1, :]
    m = jnp.max(m_w, axis=0)
    s = jnp.sum(s_w * jnp.exp(m_w - m[None, :]), axis=0)
    w_ref[...] = m + jnp.log(s + 1e-16)


_tc2_stage = pl.pallas_call(
    _tc2_body,
    out_shape=jax.ShapeDtypeStruct((NP,), jnp.float32),
)


def _sca_body(srcR, dstR, gR, s1_hbm, s2_hbm, aR, pub_hbm,
              src2, dst2, aex, m_t, s_t, buf1, buf2, shift_k, shift_v):
    cid = lax.axis_index("c")
    tid = lax.axis_index("s")
    wid = tid * 2 + cid
    inv_temp = jnp.float32(1.0 / TEMP)

    pltpu.sync_copy(srcR.at[wid], src2)
    pltpu.sync_copy(dstR.at[wid], dst2)
    pltpu.sync_copy(gR.at[wid], aex)
    pltpu.sync_copy(s1_hbm, buf1)
    pltpu.sync_copy(s2_hbm, buf2)

    def _init(i, c):
        m_t[pl.ds(i * 16, 16)] = jnp.full((16,), _NEG, jnp.float32)
        s_t[pl.ds(i * 16, 16)] = jnp.zeros((16,), jnp.float32)
        return c
    lax.fori_loop(0, NP // 16, _init, 0)

    shift_k[pl.ds(0, 16)] = jnp.full((16,), -1, jnp.int32)
    shift_v[pl.ds(0, 16)] = jnp.full((16,), _NEG, jnp.float32)

    def _p1(r, c):
        for q in range(8):
            sl = pl.ds(q * 16, 16)
            src16 = src2[r, sl]
            dst16 = dst2[r, sl]
            g16 = aex[r, sl]
            a16 = (plsc.load_gather(buf1, [src16])
                   + plsc.load_gather(buf2, [dst16]) + g16) * inv_temp
            aex[r, sl] = a16
            ks, vs = plsc.sort_key_val(src16, a16)
            shift_k[pl.ds(16, 16)] = ks
            shift_v[pl.ds(16, 16)] = vs
            for s in (1, 2, 4, 8):
                kk = shift_k[pl.ds(16 - s, 16)]
                vv = shift_v[pl.ds(16 - s, 16)]
                vs = jnp.where(kk == ks, jnp.maximum(vs, vv), vs)
                shift_v[pl.ds(16, 16)] = vs
            _u, last = plsc.scan_count(ks)
            cur = plsc.load_gather(m_t, [ks], mask=last)
            plsc.store_scatter(m_t, [ks], jnp.maximum(cur, vs), mask=last)
        return c
    lax.fori_loop(0, SROWS, _p1, 0)

    pltpu.sync_copy(aex, aR.at[wid])

    shift_v[pl.ds(0, 16)] = jnp.zeros((16,), jnp.float32)

    def _p2(r, c):
        for q in range(8):
            sl = pl.ds(q * 16, 16)
            src16 = src2[r, sl]
            ex = jnp.exp(aex[r, sl] - plsc.load_gather(m_t, [src16]))
            ks, vs = plsc.sort_key_val(src16, ex)
            shift_k[pl.ds(16, 16)] = ks
            shift_v[pl.ds(16, 16)] = vs
            for s in (1, 2, 4, 8):
                kk = shift_k[pl.ds(16 - s, 16)]
                vv = shift_v[pl.ds(16 - s, 16)]
                vs = vs + jnp.where(kk == ks, vv, jnp.float32(0.0))
                shift_v[pl.ds(16, 16)] = vs
            _u, last = plsc.scan_count(ks)
            plsc.addupdate_scatter(s_t, [ks], vs, mask=last)
        return c
    lax.fori_loop(0, SROWS, _p2, 0)

    pltpu.sync_copy(m_t, pub_hbm.at[wid, 0])
    pltpu.sync_copy(s_t, pub_hbm.at[wid, 1])


_sca_stage = pl.kernel(
    _sca_body,
    out_type=(jax.ShapeDtypeStruct((32, SROWS, 128), jnp.float32),
              jax.ShapeDtypeStruct((32, 2, NP), jnp.float32)),
    mesh=_SC_MESH,
    compiler_params=_SC_PARAMS,
    scratch_types=[
        pltpu.VMEM((SROWS, 128), jnp.int32),
        pltpu.VMEM((SROWS, 128), jnp.int32),
        pltpu.VMEM((SROWS, 128), jnp.float32),
        pltpu.VMEM((NP,), jnp.float32),
        pltpu.VMEM((NP,), jnp.float32),
        pltpu.VMEM((NP,), jnp.float32),
        pltpu.VMEM((NP,), jnp.float32),
        pltpu.VMEM((32,), jnp.int32),
        pltpu.VMEM((32,), jnp.float32),
    ],
)


def _scb_body(x2_hbm, srcRb, dstRb, aRb, w_hbm, outp_hbm,
              gidx2, dst2, aex, wbuf, rows, gsem, ssem, out_sp):
    cid = lax.axis_index("c")
    tid = lax.axis_index("s")

    pltpu.sync_copy(srcRb.at[tid], gidx2)
    pltpu.sync_copy(dstRb.at[tid], dst2)
    pltpu.sync_copy(aRb.at[tid], aex)
    pltpu.sync_copy(w_hbm, wbuf)

    def _zrows(i, c):
        for q in range(4):
            rows[0, i, pl.ds(q * 16, 16)] = jnp.zeros((16,), jnp.float32)
        return c
    lax.fori_loop(0, 128, _zrows, 0)
    for b in range(NPT // 128):
        pltpu.sync_copy(rows.at[0],
                        out_sp.at[pl.ds(tid * NPT + b * 128, 128)])

    def _prep(r, c):
        for q in range(8):
            sl = pl.ds(q * 16, 16)
            src16 = gidx2[r, sl]
            aex[r, sl] = jnp.exp(aex[r, sl]
                                 - plsc.load_gather(wbuf, [src16]))
            gidx2[r, sl] = jnp.minimum(src16 * 2 + cid, jnp.int32(2 * N - 1))
        return c
    lax.fori_loop(0, BROWS, _prep, 0)
    plsc.subcore_barrier()

    pltpu.async_copy(x2_hbm.at[gidx2.at[0]], rows.at[0], gsem)

    def _p4(r, c):
        b = r % 2
        pltpu.make_async_copy(x2_hbm.at[gidx2.at[r]], rows.at[b], gsem).wait()

        @pl.when(r + 1 < BROWS)
        def _pref():
            @pl.when(r >= 1)
            def _free():
                pltpu.make_async_copy(rows.at[1 - b],
                                      out_sp.at[dst2.at[r - 1]], ssem).wait()
            pltpu.async_copy(x2_hbm.at[gidx2.at[r + 1]], rows.at[1 - b], gsem)

        r16 = jnp.full((16,), r, jnp.int32)

        @plsc.parallel_loop(0, 128, unroll=8)
        def _scale(e):
            av = plsc.load_gather(aex, [r16, jnp.full((16,), e, jnp.int32)])
            for dq in range(4):
                dsl = pl.ds(dq * 16, 16)
                rows[b, e, dsl] = rows[b, e, dsl] * av

        pltpu.async_copy(rows.at[b], out_sp.at[dst2.at[r]], ssem, add=True)
        return c
    lax.fori_loop(0, BROWS, _p4, 0)
    for rr in (BROWS - 2, BROWS - 1):
        pltpu.make_async_copy(rows.at[rr % 2],
                              out_sp.at[dst2.at[rr]], ssem).wait()
    plsc.subcore_barrier()

    pltpu.sync_copy(out_sp.at[pl.ds(tid * NPT, NPT)],
                    outp_hbm.at[cid, pl.ds(tid * NPT, NPT)])


_scb_stage = pl.kernel(
    _scb_body,
    out_type=jax.ShapeDtypeStruct((2, NP, DH), jnp.float32),
    mesh=_SC_MESH,
    compiler_params=_SC_PARAMS,
    scratch_types=[
        pltpu.VMEM((BROWS, 128), jnp.int32),
        pltpu.VMEM((BROWS, 128), jnp.int32),
        pltpu.VMEM((BROWS, 128), jnp.float32),
        pltpu.VMEM((NP,), jnp.float32),
        pltpu.VMEM((2, 128, DH), jnp.float32),
        pltpu.SemaphoreType.DMA,
        pltpu.SemaphoreType.DMA,
        pltpu.VMEM_SHARED((NP, DH), jnp.float32),
    ],
)


@jax.jit
def kernel(x, edge_index, batch, att_weight):
    src = edge_index[0]
    dst = edge_index[1]
    x2 = x.reshape(N * 2, DH)

    pad_ids = (N + (jnp.arange(EP - E, dtype=jnp.int32) % 8)).astype(jnp.int32)
    srcR = jnp.concatenate([src, pad_ids]).reshape(32, SROWS, 128)
    dstR = jnp.concatenate([dst, pad_ids]).reshape(32, SROWS, 128)
    srcRb = srcR.reshape(16, BROWS, 128)
    dstRb = dstR.reshape(16, BROWS, 128)

    u = jax.random.uniform(jax.random.key(42), (E,), dtype=jnp.float32)
    up = jnp.concatenate([u, jnp.full((EP - E,), 0.5, jnp.float32)])
    up = up.reshape(UROWS, 128)

    w1 = att_weight[0, :D]
    w2 = att_weight[0, D:]
    Wp = jnp.zeros((D, 8), jnp.float32).at[:, 0].set(w1).at[:, 1].set(w2)

    s_pad, g2 = _tc1_stage(x, Wp, up)
    gR = g2.reshape(32, SROWS, 128)

    s1p = jnp.pad(s_pad[:, 0], (0, NP - N))
    s2p = jnp.pad(s_pad[:, 1], (0, NP - N))
    aR, pub = _sca_stage(srcR, dstR, gR, s1p, s2p)
    wn = _tc2_stage(pub)
    aRb = aR.reshape(16, BROWS, 128)
    outp = _scb_stage(x2, srcRb, dstRb, aRb, wn)
    out = outp.transpose(1, 0, 2).reshape(NP, D)[:N]

    score = jnp.zeros((N,), out.dtype)
    perm = jnp.arange(N, dtype=jnp.int32)
    return (out, edge_index, batch, perm, score)


# speedup vs baseline: 1.3177x; 1.3177x over previous
import jax
import jax.numpy as jnp
from jax import lax
from jax.experimental import pallas as pl
from jax.experimental.pallas import tpu as pltpu
from jax.experimental.pallas import tpu_sc as plsc

N = 10000
E = 320000
D = 128
TEMP = 0.1
EPS = 1e-20

NP = 10240
SROWS = 79
EP = 32 * SROWS * 128
BROWS = 2 * SROWS
NPT = NP // 16
DH = D // 2
UROWS = EP // 128

_NEG = -1e30

_SC_PARAMS = pltpu.CompilerParams(needs_layout_passes=False,
                                  use_tc_tiling_on_sc=False)
_SC_MESH = plsc.VectorSubcoreMesh(core_axis_name="c", subcore_axis_name="s")


def _tc1_body(x_ref, w_ref, u_ref, s_ref, g_ref):
    s_ref[...] = jnp.dot(x_ref[...], w_ref[...],
                         preferred_element_type=jnp.float32)
    u = u_ref[...]
    g_ref[...] = -jnp.log(-jnp.log(u + EPS) + EPS)


_tc1_stage = pl.pallas_call(
    _tc1_body,
    out_shape=(
        jax.ShapeDtypeStruct((N, 8), jnp.float32),
        jax.ShapeDtypeStruct((UROWS, 128), jnp.float32),
    ),
)


def _tc2_body(pub_ref, w_ref):
    m_w = pub_ref[:, 0, :]
    s_w = pub_ref[:, 1, :]
    m = jnp.max(m_w, axis=0)
    s = jnp.sum(s_w * jnp.exp(m_w - m[None, :]), axis=0)
    w_ref[...] = m + jnp.log(s + 1e-16)


_tc2_stage = pl.pallas_call(
    _tc2_body,
    out_shape=jax.ShapeDtypeStruct((NP,), jnp.float32),
)


def _sca_body(srcR, dstR, gR, s1_hbm, s2_hbm, aR, pub_hbm,
              src2, dst2, aex, m_t, s_t, buf1, buf2, shift_k, shift_v):
    cid = lax.axis_index("c")
    tid = lax.axis_index("s")
    wid = tid * 2 + cid
    inv_temp = jnp.float32(1.0 / TEMP)

    pltpu.sync_copy(srcR.at[wid], src2)
    pltpu.sync_copy(dstR.at[wid], dst2)
    pltpu.sync_copy(gR.at[wid], aex)
    pltpu.sync_copy(s1_hbm, buf1)
    pltpu.sync_copy(s2_hbm, buf2)

    def _init(i, c):
        m_t[pl.ds(i * 16, 16)] = jnp.full((16,), _NEG, jnp.float32)
        s_t[pl.ds(i * 16, 16)] = jnp.zeros((16,), jnp.float32)
        return c
    lax.fori_loop(0, NP // 16, _init, 0)

    shift_k[pl.ds(0, 16)] = jnp.full((16,), -1, jnp.int32)
    shift_v[pl.ds(0, 16)] = jnp.full((16,), _NEG, jnp.float32)

    def _p1(r, c):
        for q in range(8):
            sl = pl.ds(q * 16, 16)
            src16 = src2[r, sl]
            dst16 = dst2[r, sl]
            g16 = aex[r, sl]
            a16 = (plsc.load_gather(buf1, [src16])
                   + plsc.load_gather(buf2, [dst16]) + g16) * inv_temp
            aex[r, sl] = a16
            ks, vs = plsc.sort_key_val(src16, a16)
            shift_k[pl.ds(16, 16)] = ks
            shift_v[pl.ds(16, 16)] = vs
            for s in (1, 2, 4, 8):
                kk = shift_k[pl.ds(16 - s, 16)]
                vv = shift_v[pl.ds(16 - s, 16)]
                vs = jnp.where(kk == ks, jnp.maximum(vs, vv), vs)
                shift_v[pl.ds(16, 16)] = vs
            _u, last = plsc.scan_count(ks)
            cur = plsc.load_gather(m_t, [ks], mask=last)
            plsc.store_scatter(m_t, [ks], jnp.maximum(cur, vs), mask=last)
        return c
    lax.fori_loop(0, SROWS, _p1, 0)

    pltpu.sync_copy(aex, aR.at[wid])

    shift_v[pl.ds(0, 16)] = jnp.zeros((16,), jnp.float32)

    def _p2(r, c):
        for q in range(8):
            sl = pl.ds(q * 16, 16)
            src16 = src2[r, sl]
            ex = jnp.exp(aex[r, sl] - plsc.load_gather(m_t, [src16]))
            ks, vs = plsc.sort_key_val(src16, ex)
            shift_k[pl.ds(16, 16)] = ks
            shift_v[pl.ds(16, 16)] = vs
            for s in (1, 2, 4, 8):
                kk = shift_k[pl.ds(16 - s, 16)]
                vv = shift_v[pl.ds(16 - s, 16)]
                vs = vs + jnp.where(kk == ks, vv, jnp.float32(0.0))
                shift_v[pl.ds(16, 16)] = vs
            _u, last = plsc.scan_count(ks)
            plsc.addupdate_scatter(s_t, [ks], vs, mask=last)
        return c
    lax.fori_loop(0, SROWS, _p2, 0)

    pltpu.sync_copy(m_t, pub_hbm.at[wid, 0])
    pltpu.sync_copy(s_t, pub_hbm.at[wid, 1])


_sca_stage = pl.kernel(
    _sca_body,
    out_type=(jax.ShapeDtypeStruct((32, SROWS, 128), jnp.float32),
              jax.ShapeDtypeStruct((32, 2, NP), jnp.float32)),
    mesh=_SC_MESH,
    compiler_params=_SC_PARAMS,
    scratch_types=[
        pltpu.VMEM((SROWS, 128), jnp.int32),
        pltpu.VMEM((SROWS, 128), jnp.int32),
        pltpu.VMEM((SROWS, 128), jnp.float32),
        pltpu.VMEM((NP,), jnp.float32),
        pltpu.VMEM((NP,), jnp.float32),
        pltpu.VMEM((NP,), jnp.float32),
        pltpu.VMEM((NP,), jnp.float32),
        pltpu.VMEM((32,), jnp.int32),
        pltpu.VMEM((32,), jnp.float32),
    ],
)


def _scb_body(x2_hbm, srcRb, dstRb, aRb, w_hbm, outp_hbm,
              gidx2, dst2, aex, wbuf, rows, gsem, ssem, out_sp):
    cid = lax.axis_index("c")
    tid = lax.axis_index("s")

    pltpu.sync_copy(srcRb.at[tid], gidx2)
    pltpu.sync_copy(dstRb.at[tid], dst2)
    pltpu.sync_copy(aRb.at[tid], aex)
    pltpu.sync_copy(w_hbm, wbuf)

    def _zrows(i, c):
        for q in range(4):
            rows[0, i, pl.ds(q * 16, 16)] = jnp.zeros((16,), jnp.float32)
        return c
    lax.fori_loop(0, 128, _zrows, 0)
    for b in range(NPT // 128):
        pltpu.sync_copy(rows.at[0],
                        out_sp.at[pl.ds(tid * NPT + b * 128, 128)])

    def _prep(r, c):
        for q in range(8):
            sl = pl.ds(q * 16, 16)
            src16 = gidx2[r, sl]
            aex[r, sl] = jnp.exp(aex[r, sl]
                                 - plsc.load_gather(wbuf, [src16]))
            gidx2[r, sl] = jnp.where(src16 < N, src16 * 2 + cid,
                                     src16 - N)
        return c
    lax.fori_loop(0, BROWS, _prep, 0)
    plsc.subcore_barrier()

    pltpu.async_copy(x2_hbm.at[gidx2.at[0]], rows.at[0], gsem)

    def _p4(r, c):
        b = r % 2
        pltpu.make_async_copy(x2_hbm.at[gidx2.at[r]], rows.at[b], gsem).wait()

        @pl.when(r + 1 < BROWS)
        def _pref():
            @pl.when(r >= 1)
            def _free():
                pltpu.make_async_copy(rows.at[1 - b],
                                      out_sp.at[dst2.at[r - 1]], ssem).wait()
            pltpu.async_copy(x2_hbm.at[gidx2.at[r + 1]], rows.at[1 - b], gsem)

        r16 = jnp.full((16,), r, jnp.int32)

        @plsc.parallel_loop(0, 128, unroll=8)
        def _scale(e):
            av = plsc.load_gather(aex, [r16, jnp.full((16,), e, jnp.int32)])
            for dq in range(4):
                dsl = pl.ds(dq * 16, 16)
                rows[b, e, dsl] = rows[b, e, dsl] * av

        pltpu.async_copy(rows.at[b], out_sp.at[dst2.at[r]], ssem, add=True)
        return c
    lax.fori_loop(0, BROWS, _p4, 0)
    for rr in (BROWS - 2, BROWS - 1):
        pltpu.make_async_copy(rows.at[rr % 2],
                              out_sp.at[dst2.at[rr]], ssem).wait()
    plsc.subcore_barrier()

    pltpu.sync_copy(out_sp.at[pl.ds(tid * NPT, NPT)],
                    outp_hbm.at[cid, pl.ds(tid * NPT, NPT)])


_scb_stage = pl.kernel(
    _scb_body,
    out_type=jax.ShapeDtypeStruct((2, NP, DH), jnp.float32),
    mesh=_SC_MESH,
    compiler_params=_SC_PARAMS,
    scratch_types=[
        pltpu.VMEM((BROWS, 128), jnp.int32),
        pltpu.VMEM((BROWS, 128), jnp.int32),
        pltpu.VMEM((BROWS, 128), jnp.float32),
        pltpu.VMEM((NP,), jnp.float32),
        pltpu.VMEM((2, 128, DH), jnp.float32),
        pltpu.SemaphoreType.DMA,
        pltpu.SemaphoreType.DMA,
        pltpu.VMEM_SHARED((NP, DH), jnp.float32),
    ],
)


@jax.jit
def kernel(x, edge_index, batch, att_weight):
    src = edge_index[0]
    dst = edge_index[1]
    x2 = x.reshape(N * 2, DH)

    pad_ids = (N + (jnp.arange(EP - E, dtype=jnp.int32)
                    % (NP - N))).astype(jnp.int32)
    srcR = jnp.concatenate([src, pad_ids]).reshape(32, SROWS, 128)
    dstR = jnp.concatenate([dst, pad_ids]).reshape(32, SROWS, 128)
    srcRb = srcR.reshape(16, BROWS, 128)
    dstRb = dstR.reshape(16, BROWS, 128)

    u = jax.random.uniform(jax.random.key(42), (E,), dtype=jnp.float32)
    up = jnp.concatenate([u, jnp.full((EP - E,), 0.5, jnp.float32)])
    up = up.reshape(UROWS, 128)

    w1 = att_weight[0, :D]
    w2 = att_weight[0, D:]
    Wp = jnp.zeros((D, 8), jnp.float32).at[:, 0].set(w1).at[:, 1].set(w2)

    s_pad, g2 = _tc1_stage(x, Wp, up)
    gR = g2.reshape(32, SROWS, 128)

    s1p = jnp.pad(s_pad[:, 0], (0, NP - N))
    s2p = jnp.pad(s_pad[:, 1], (0, NP - N))
    aR, pub = _sca_stage(srcR, dstR, gR, s1p, s2p)
    wn = _tc2_stage(pub)
    aRb = aR.reshape(16, BROWS, 128)
    outp = _scb_stage(x2, srcRb, dstRb, aRb, wn)
    out = outp.transpose(1, 0, 2).reshape(NP, D)[:N]

    score = jnp.zeros((N,), out.dtype)
    perm = jnp.arange(N, dtype=jnp.int32)
    return (out, edge_index, batch, perm, score)


# speedup vs baseline: 1.3928x; 1.0569x over previous
import jax
import jax.numpy as jnp
from jax import lax
from jax.experimental import pallas as pl
from jax.experimental.pallas import tpu as pltpu
from jax.experimental.pallas import tpu_sc as plsc

N = 10000
E = 320000
D = 128
TEMP = 0.1
EPS = 1e-20

NP = 10240
SROWS = 79
EP = 32 * SROWS * 128
BROWS = 2 * SROWS
NPT = NP // 16
DH = D // 2
UROWS = EP // 128

_NEG = -1e30

_SC_PARAMS = pltpu.CompilerParams(needs_layout_passes=False,
                                  use_tc_tiling_on_sc=False)
_SC_MESH = plsc.VectorSubcoreMesh(core_axis_name="c", subcore_axis_name="s")


_G_HALF = 0.36651292
_EROWS = E // 128
_PROWS = UROWS - _EROWS


def _tc1_body(x_ref, w_ref, u_ref, ei_ref, s1_ref, s2_ref, g_ref,
              src_ref, dst_ref):
    x = x_ref[...]
    s1_ref[pl.ds(0, N)] = jnp.sum(x * w_ref[0][None, :], axis=1)
    s2_ref[pl.ds(0, N)] = jnp.sum(x * w_ref[1][None, :], axis=1)
    zpad = jnp.zeros((NP - N,), jnp.float32)
    s1_ref[pl.ds(N, NP - N)] = zpad
    s2_ref[pl.ds(N, NP - N)] = zpad

    u = u_ref[...]
    g_ref[pl.ds(0, _EROWS), :] = -jnp.log(-jnp.log(u + EPS) + EPS)
    g_ref[pl.ds(_EROWS, _PROWS), :] = jnp.full((_PROWS, 128), _G_HALF,
                                               jnp.float32)
    src_ref[pl.ds(0, _EROWS), :] = ei_ref[0].reshape(_EROWS, 128)
    dst_ref[pl.ds(0, _EROWS), :] = ei_ref[1].reshape(_EROWS, 128)
    ii = (lax.broadcasted_iota(jnp.int32, (_PROWS, 128), 0) * 128
          + lax.broadcasted_iota(jnp.int32, (_PROWS, 128), 1))
    pad_ids = N + ii % (NP - N)
    src_ref[pl.ds(_EROWS, _PROWS), :] = pad_ids
    dst_ref[pl.ds(_EROWS, _PROWS), :] = pad_ids


_tc1_stage = pl.pallas_call(
    _tc1_body,
    out_shape=(
        jax.ShapeDtypeStruct((NP,), jnp.float32),
        jax.ShapeDtypeStruct((NP,), jnp.float32),
        jax.ShapeDtypeStruct((UROWS, 128), jnp.float32),
        jax.ShapeDtypeStruct((UROWS, 128), jnp.int32),
        jax.ShapeDtypeStruct((UROWS, 128), jnp.int32),
    ),
)


def _tc2_body(pub_ref, w_ref):
    m_w = pub_ref[:, 0, :]
    s_w = pub_ref[:, 1, :]
    m = jnp.max(m_w, axis=0)
    s = jnp.sum(s_w * jnp.exp(m_w - m[None, :]), axis=0)
    w_ref[...] = m + jnp.log(s + 1e-16)


_tc2_stage = pl.pallas_call(
    _tc2_body,
    out_shape=jax.ShapeDtypeStruct((NP,), jnp.float32),
)


def _sca_body(srcR, dstR, gR, s1_hbm, s2_hbm, aR, pub_hbm,
              src2, dst2, aex, m_t, s_t, buf1, buf2, shift_k, shift_v):
    cid = lax.axis_index("c")
    tid = lax.axis_index("s")
    wid = tid * 2 + cid
    inv_temp = jnp.float32(1.0 / TEMP)

    pltpu.sync_copy(srcR.at[wid], src2)
    pltpu.sync_copy(dstR.at[wid], dst2)
    pltpu.sync_copy(gR.at[wid], aex)
    pltpu.sync_copy(s1_hbm, buf1)
    pltpu.sync_copy(s2_hbm, buf2)

    def _init(i, c):
        m_t[pl.ds(i * 16, 16)] = jnp.full((16,), _NEG, jnp.float32)
        s_t[pl.ds(i * 16, 16)] = jnp.zeros((16,), jnp.float32)
        return c
    lax.fori_loop(0, NP // 16, _init, 0)

    shift_k[pl.ds(0, 16)] = jnp.full((16,), -1, jnp.int32)
    shift_v[pl.ds(0, 16)] = jnp.full((16,), _NEG, jnp.float32)

    def _p1(r, c):
        for q in range(8):
            sl = pl.ds(q * 16, 16)
            src16 = src2[r, sl]
            dst16 = dst2[r, sl]
            g16 = aex[r, sl]
            a16 = (plsc.load_gather(buf1, [src16])
                   + plsc.load_gather(buf2, [dst16]) + g16) * inv_temp
            aex[r, sl] = a16
            ks, vs = plsc.sort_key_val(src16, a16)
            shift_k[pl.ds(16, 16)] = ks
            shift_v[pl.ds(16, 16)] = vs
            for s in (1, 2, 4, 8):
                kk = shift_k[pl.ds(16 - s, 16)]
                vv = shift_v[pl.ds(16 - s, 16)]
                vs = jnp.where(kk == ks, jnp.maximum(vs, vv), vs)
                shift_v[pl.ds(16, 16)] = vs
            _u, last = plsc.scan_count(ks)
            cur = plsc.load_gather(m_t, [ks], mask=last)
            plsc.store_scatter(m_t, [ks], jnp.maximum(cur, vs), mask=last)
        return c
    lax.fori_loop(0, SROWS, _p1, 0)

    pltpu.sync_copy(aex, aR.at[wid])

    shift_v[pl.ds(0, 16)] = jnp.zeros((16,), jnp.float32)

    def _p2(r, c):
        for q in range(8):
            sl = pl.ds(q * 16, 16)
            src16 = src2[r, sl]
            ex = jnp.exp(aex[r, sl] - plsc.load_gather(m_t, [src16]))
            ks, vs = plsc.sort_key_val(src16, ex)
            shift_k[pl.ds(16, 16)] = ks
            shift_v[pl.ds(16, 16)] = vs
            for s in (1, 2, 4, 8):
                kk = shift_k[pl.ds(16 - s, 16)]
                vv = shift_v[pl.ds(16 - s, 16)]
                vs = vs + jnp.where(kk == ks, vv, jnp.float32(0.0))
                shift_v[pl.ds(16, 16)] = vs
            _u, last = plsc.scan_count(ks)
            plsc.addupdate_scatter(s_t, [ks], vs, mask=last)
        return c
    lax.fori_loop(0, SROWS, _p2, 0)

    pltpu.sync_copy(m_t, pub_hbm.at[wid, 0])
    pltpu.sync_copy(s_t, pub_hbm.at[wid, 1])


_sca_stage = pl.kernel(
    _sca_body,
    out_type=(jax.ShapeDtypeStruct((32, SROWS, 128), jnp.float32),
              jax.ShapeDtypeStruct((32, 2, NP), jnp.float32)),
    mesh=_SC_MESH,
    compiler_params=_SC_PARAMS,
    scratch_types=[
        pltpu.VMEM((SROWS, 128), jnp.int32),
        pltpu.VMEM((SROWS, 128), jnp.int32),
        pltpu.VMEM((SROWS, 128), jnp.float32),
        pltpu.VMEM((NP,), jnp.float32),
        pltpu.VMEM((NP,), jnp.float32),
        pltpu.VMEM((NP,), jnp.float32),
        pltpu.VMEM((NP,), jnp.float32),
        pltpu.VMEM((32,), jnp.int32),
        pltpu.VMEM((32,), jnp.float32),
    ],
)


def _scb_body(x2_hbm, srcRb, dstRb, aRb, w_hbm, outp_hbm,
              gidx2, dst2, aex, wbuf, rows, gsem, ssem, out_sp):
    cid = lax.axis_index("c")
    tid = lax.axis_index("s")

    pltpu.sync_copy(srcRb.at[tid], gidx2)
    pltpu.sync_copy(dstRb.at[tid], dst2)
    pltpu.sync_copy(aRb.at[tid], aex)
    pltpu.sync_copy(w_hbm, wbuf)

    def _zrows(i, c):
        for q in range(4):
            rows[0, i, pl.ds(q * 16, 16)] = jnp.zeros((16,), jnp.float32)
        return c
    lax.fori_loop(0, 128, _zrows, 0)
    for b in range(NPT // 128):
        pltpu.sync_copy(rows.at[0],
                        out_sp.at[pl.ds(tid * NPT + b * 128, 128)])

    def _prep(r, c):
        for q in range(8):
            sl = pl.ds(q * 16, 16)
            src16 = gidx2[r, sl]
            aex[r, sl] = jnp.exp(aex[r, sl]
                                 - plsc.load_gather(wbuf, [src16]))
            gidx2[r, sl] = jnp.where(src16 < N, src16 * 2 + cid,
                                     src16 - N)
        return c
    lax.fori_loop(0, BROWS, _prep, 0)
    plsc.subcore_barrier()

    pltpu.async_copy(x2_hbm.at[gidx2.at[0]], rows.at[0], gsem)

    def _p4(r, c):
        b = r % 2
        pltpu.make_async_copy(x2_hbm.at[gidx2.at[r]], rows.at[b], gsem).wait()

        @pl.when(r + 1 < BROWS)
        def _pref():
            @pl.when(r >= 1)
            def _free():
                pltpu.make_async_copy(rows.at[1 - b],
                                      out_sp.at[dst2.at[r - 1]], ssem).wait()
            pltpu.async_copy(x2_hbm.at[gidx2.at[r + 1]], rows.at[1 - b], gsem)

        r16 = jnp.full((16,), r, jnp.int32)

        @plsc.parallel_loop(0, 128, unroll=8)
        def _scale(e):
            av = plsc.load_gather(aex, [r16, jnp.full((16,), e, jnp.int32)])
            for dq in range(4):
                dsl = pl.ds(dq * 16, 16)
                rows[b, e, dsl] = rows[b, e, dsl] * av

        pltpu.async_copy(rows.at[b], out_sp.at[dst2.at[r]], ssem, add=True)
        return c
    lax.fori_loop(0, BROWS, _p4, 0)
    for rr in (BROWS - 2, BROWS - 1):
        pltpu.make_async_copy(rows.at[rr % 2],
                              out_sp.at[dst2.at[rr]], ssem).wait()
    plsc.subcore_barrier()

    pltpu.sync_copy(out_sp.at[pl.ds(tid * NPT, NPT)],
                    outp_hbm.at[cid, pl.ds(tid * NPT, NPT)])


_scb_stage = pl.kernel(
    _scb_body,
    out_type=jax.ShapeDtypeStruct((2, NP, DH), jnp.float32),
    mesh=_SC_MESH,
    compiler_params=_SC_PARAMS,
    scratch_types=[
        pltpu.VMEM((BROWS, 128), jnp.int32),
        pltpu.VMEM((BROWS, 128), jnp.int32),
        pltpu.VMEM((BROWS, 128), jnp.float32),
        pltpu.VMEM((NP,), jnp.float32),
        pltpu.VMEM((2, 128, DH), jnp.float32),
        pltpu.SemaphoreType.DMA,
        pltpu.SemaphoreType.DMA,
        pltpu.VMEM_SHARED((NP, DH), jnp.float32),
    ],
)


@jax.jit
def kernel(x, edge_index, batch, att_weight):
    x2 = x.reshape(N * 2, DH)

    u = jax.random.uniform(jax.random.key(42), (E,), dtype=jnp.float32)
    u2 = u.reshape(_EROWS, 128)
    w12 = att_weight.reshape(2, D)

    s1p, s2p, g2, srcF, dstF = _tc1_stage(x, w12, u2, edge_index)
    gR = g2.reshape(32, SROWS, 128)
    srcR = srcF.reshape(32, SROWS, 128)
    dstR = dstF.reshape(32, SROWS, 128)

    aR, pub = _sca_stage(srcR, dstR, gR, s1p, s2p)
    wn = _tc2_stage(pub)
    aRb = aR.reshape(16, BROWS, 128)
    outp = _scb_stage(x2, srcF.reshape(16, BROWS, 128),
                      dstF.reshape(16, BROWS, 128), aRb, wn)
    out = outp.transpose(1, 0, 2).reshape(NP, D)[:N]

    score = jnp.zeros((N,), out.dtype)
    perm = jnp.arange(N, dtype=jnp.int32)
    return (out, edge_index, batch, perm, score)
